# baseline (speedup 1.0000x reference)
import jax
import jax.numpy as jnp
from jax import lax
from jax.experimental import pallas as pl
from jax.experimental.pallas import tpu as pltpu
from jax.experimental.pallas import tpu_sc as plsc

N = 50000
E = 800000
G = 1024
H = 256
OUT = 9

NC = 2
NS = 16
TILE_NODES = 3136
NPAD = NS * TILE_NODES
SUB = 784
NSUB = TILE_NODES // SUB
GROUPS = SUB // 16
EPT = E // NS
ECH = 2000
NCHUNK = EPT // ECH
GP = G + 64
WP = 16


def _prep_body(a_ref, b_ref, c_ref, d_ref, bt_ref,
               xe_ref, xb_ref, xd_ref, xw_ref, xh_ref,
               sa_ref, da_ref, sb_ref, db_ref,
               sc_ref, dc_ref, sd_ref, dd_ref, bto_ref,
               te_ref, tb_ref, td_ref, tw_ref, xho_ref, z8_ref):
    sa_ref[...] = a_ref[0, :]
    da_ref[...] = a_ref[1, :]
    sb_ref[...] = b_ref[0, :]
    db_ref[...] = b_ref[1, :]
    sc_ref[...] = c_ref[0, :]
    dc_ref[...] = c_ref[1, :]
    sd_ref[...] = d_ref[0, :]
    dd_ref[...] = d_ref[1, :]
    nb = bt_ref.shape[0]
    i = pl.program_id(0)
    gidx = lax.broadcasted_iota(jnp.int32, (nb,), 0) + i * nb
    bto_ref[...] = jnp.where(gidx < N, bt_ref[...], G)
    for tref, xref, d in ((te_ref, xe_ref, 4), (tb_ref, xb_ref, 3),
                          (td_ref, xd_ref, 4), (tw_ref, xw_ref, 4)):
        tref[:, 0:d] = xref[...]
        tref[:, d:d + 1] = jnp.ones((nb, 1), jnp.float32)
        tref[:, d + 1:8] = jnp.zeros((nb, 7 - d), jnp.float32)
    gidx2 = lax.broadcasted_iota(jnp.int32, (nb, 6), 0) + i * nb
    xho_ref[...] = jnp.where(gidx2 < N, xh_ref[...], 0.0)
    z8_ref[...] = jnp.zeros_like(z8_ref)


def _sc_body(xe8, xb8, xd8, xw8, xh, batchp,
             sA0, dA0, sB0, dB0, sA1, dA1, sB1, dB1, z8, zp, out,
             accA, accB, pooled, idx0, idx1, dstb, rows0, rows1,
             locA, locB, locH, bidx, rowbuf, gsem0, gsem1):
    c = lax.axis_index("c")
    s = lax.axis_index("s")

    r0 = s * TILE_NODES
    pltpu.sync_copy(z8.at[pl.ds(r0, TILE_NODES)], accA.at[pl.ds(r0, TILE_NODES)])
    pltpu.sync_copy(z8.at[pl.ds(r0, TILE_NODES)], accB.at[pl.ds(r0, TILE_NODES)])

    @pl.when(s == 0)
    def _():
        pltpu.sync_copy(zp, pooled)

    plsc.subcore_barrier()

    ebase = s * EPT

    def _edges(src, dst, table, acc):
        idxs = (idx0, idx1)
        rows = (rows0, rows1)
        gsems = (gsem0, gsem1)
        desc = [None, None]
        pltpu.sync_copy(src.at[pl.ds(ebase, ECH)], idx0)
        desc[0] = pltpu.async_copy(table.at[idx0], rows0, gsem0)
        for i in range(NCHUNK):
            p = i % 2
            q = 1 - p
            if i + 1 < NCHUNK:
                b = ebase + (i + 1) * ECH
                pltpu.sync_copy(src.at[pl.ds(b, ECH)], idxs[q])
                desc[q] = pltpu.async_copy(table.at[idxs[q]], rows[q], gsems[q])
            desc[p].wait()
            pltpu.sync_copy(dst.at[pl.ds(ebase + i * ECH, ECH)], dstb)
            pltpu.sync_copy(rows[p], acc.at[dstb], add=True)

    @pl.when(c == 0)
    def _():
        _edges(sA0, dA0, xe8, accA)
        _edges(sB0, dB0, xb8, accB)

    @pl.when(c == 1)
    def _():
        _edges(sA1, dA1, xd8, accA)
        _edges(sB1, dB1, xw8, accB)

    plsc.subcore_barrier()

    iota16 = lax.iota(jnp.int32, 16)
    colBcnt = jnp.where(c == 0, 3, 4)
    hero_f = jnp.where(c == 0, 0.0, 1.0)
    cnt_f = jnp.where(c == 0, 1.0, 0.0)

    for sub in range(NSUB):
        nb = s * TILE_NODES + sub * SUB
        pltpu.sync_copy(accA.at[pl.ds(nb, SUB)], locA)
        pltpu.sync_copy(accB.at[pl.ds(nb, SUB)], locB)
        pltpu.sync_copy(xh.at[pl.ds(nb, SUB)], locH)
        pltpu.sync_copy(batchp.at[pl.ds(nb, SUB)], bidx)

        def group(j, carry):
            r = iota16 + j * 16
            cA = plsc.load_gather(locA, [r, jnp.full((16,), 4, jnp.int32)])
            cB = plsc.load_gather(locB, [r, jnp.full((16,), colBcnt, jnp.int32)])
            sA = 1.0 / jnp.maximum(cA, 1.0)
            sB = 1.0 / jnp.maximum(cB, 1.0)
            for k in range(4):
                col = jnp.full((16,), k, jnp.int32)
                vA = plsc.load_gather(locA, [r, col])
                plsc.store_scatter(rowbuf, [r, col], vA * sA)
            for k in range(4):
                vB = plsc.load_gather(locB, [r, jnp.full((16,), k, jnp.int32)])
                plsc.store_scatter(rowbuf, [r, jnp.full((16,), 4 + k, jnp.int32)], vB * sB)
            hf = jnp.full((16,), hero_f, jnp.float32)
            for k in range(6):
                vH = plsc.load_gather(locH, [r, jnp.full((16,), k, jnp.int32)])
                plsc.store_scatter(rowbuf, [r, jnp.full((16,), 8 + k, jnp.int32)], vH * hf)
            plsc.store_scatter(rowbuf, [r, jnp.full((16,), 14, jnp.int32)],
                               jnp.full((16,), cnt_f, jnp.float32))
            plsc.store_scatter(rowbuf, [r, jnp.full((16,), 15, jnp.int32)],
                               jnp.zeros((16,), jnp.float32))
            return carry

        lax.fori_loop(0, GROUPS, group, 0)
        pltpu.sync_copy(rowbuf, pooled.at[bidx], add=True)

    plsc.subcore_barrier()

    @pl.when(s == 0)
    def _():
        pltpu.sync_copy(pooled.at[pl.ds(0, G)], out.at[c])


def _tc_body(p0_ref, p1_ref, w0_ref, w1_ref, bt_ref, wfc_ref, bfc_ref, o_ref):
    p0 = p0_ref[...]
    p1 = p1_ref[...]
    lane = lax.broadcasted_iota(jnp.int32, (G, WP), 1)
    cnt = jnp.sum(jnp.where(lane == 14, p0, 0.0), axis=1, keepdims=True)
    scale = 1.0 / jnp.maximum(cnt, 1.0)
    ind = jnp.where(cnt > 0, 1.0, 0.0)
    h = jnp.dot(p0 * scale, w0_ref[...], preferred_element_type=jnp.float32)
    h = h + jnp.dot(p1 * scale, w1_ref[...], preferred_element_type=jnp.float32)
    h = h + ind * bt_ref[...]
    o_ref[...] = jnp.dot(h, wfc_ref[...], preferred_element_type=jnp.float32) + bfc_ref[...]


def kernel(x_hero, x_enemy, x_bullet, x_door, x_wall,
           ei_def, ei_dod, ei_togo, ei_sees, ei_rdef, ei_rdod, ei_rtogo, ei_rsees,
           batch,
           Wl_def, Wr_def, b_def, Wl_dod, Wr_dod, b_dod,
           Wl_togo, Wr_togo, b_togo, Wl_sees, Wr_sees, b_sees,
           Wl_rdef, Wr_rdef, b_rdef, Wl_rdod, Wr_rdod, b_rdod,
           Wl_rtogo, Wr_rtogo, b_rtogo, Wl_rsees, Wr_rsees, b_rsees,
           Wfc, bfc):
    f32 = jnp.float32

    EBLK = 65536
    NB = 4096
    prep = pl.pallas_call(
        _prep_body,
        grid=(13,),
        in_specs=[pl.BlockSpec((2, EBLK), lambda i: (0, i))] * 4 + [
            pl.BlockSpec((NB,), lambda i: (i,)),
            pl.BlockSpec((NB, 4), lambda i: (i, 0)),
            pl.BlockSpec((NB, 3), lambda i: (i, 0)),
            pl.BlockSpec((NB, 4), lambda i: (i, 0)),
            pl.BlockSpec((NB, 4), lambda i: (i, 0)),
            pl.BlockSpec((NB, 6), lambda i: (i, 0)),
        ],
        out_specs=[pl.BlockSpec((EBLK,), lambda i: (i,))] * 8 + [
            pl.BlockSpec((NB,), lambda i: (i,)),
        ] + [pl.BlockSpec((NB, 8), lambda i: (i, 0))] * 4 + [
            pl.BlockSpec((NB, 6), lambda i: (i, 0)),
            pl.BlockSpec((NB, 8), lambda i: (i, 0)),
        ],
        out_shape=[jax.ShapeDtypeStruct((E,), jnp.int32)] * 8 + [
            jax.ShapeDtypeStruct((NPAD,), jnp.int32),
        ] + [jax.ShapeDtypeStruct((NPAD, 8), f32)] * 4 + [
            jax.ShapeDtypeStruct((NPAD, 6), f32),
            jax.ShapeDtypeStruct((NPAD, 8), f32),
        ],
    )(ei_rdef, ei_rtogo, ei_rdod, ei_rsees, batch,
      x_enemy, x_bullet, x_door, x_wall, x_hero)
    (sA0, dA0, sA1, dA1, sB0, dB0, sB1, dB1, batchp,
     xe8, xb8, xd8, xw8, xh, z8) = prep

    zp = jnp.zeros((GP, WP), f32)

    sc_fn = pl.kernel(
        _sc_body,
        out_type=jax.ShapeDtypeStruct((NC, G, WP), f32),
        mesh=plsc.VectorSubcoreMesh(core_axis_name="c", subcore_axis_name="s"),
        compiler_params=pltpu.CompilerParams(
            needs_layout_passes=False, use_tc_tiling_on_sc=False),
        scratch_types=[
            pltpu.VMEM_SHARED((NPAD, 8), f32),
            pltpu.VMEM_SHARED((NPAD, 8), f32),
            pltpu.VMEM_SHARED((GP, WP), f32),
            pltpu.VMEM((ECH,), jnp.int32),
            pltpu.VMEM((ECH,), jnp.int32),
            pltpu.VMEM((ECH,), jnp.int32),
            pltpu.VMEM((ECH, 8), f32),
            pltpu.VMEM((ECH, 8), f32),
            pltpu.VMEM((SUB, 8), f32),
            pltpu.VMEM((SUB, 8), f32),
            pltpu.VMEM((SUB, 6), f32),
            pltpu.VMEM((SUB,), jnp.int32),
            pltpu.VMEM((SUB, WP), f32),
            pltpu.SemaphoreType.DMA,
            pltpu.SemaphoreType.DMA,
        ],
    )
    parts = sc_fn(xe8, xb8, xd8, xw8,
                  xh, batchp, sA0, dA0, sB0, dB0, sA1, dA1, sB1, dB1,
                  z8, zp)

    W0 = jnp.concatenate([Wl_rdef, Wl_rdod, jnp.zeros((9, H), f32)], axis=0)
    W1 = jnp.concatenate(
        [Wl_rtogo, Wl_rsees, Wr_rdef + Wr_rdod + Wr_rtogo + Wr_rsees,
         jnp.zeros((2, H), f32)], axis=0)
    btot = (b_rdef + b_rdod + b_rtogo + b_rsees).reshape(1, H)

    q = pl.pallas_call(
        _tc_body,
        out_shape=jax.ShapeDtypeStruct((G, OUT), f32),
    )(parts[0], parts[1], W0, W1, btot, Wfc, bfc.reshape(1, OUT))
    return q


# speedup vs baseline: 1.2392x; 1.2392x over previous
import jax
import jax.numpy as jnp
from jax import lax
from jax.experimental import pallas as pl
from jax.experimental.pallas import tpu as pltpu
from jax.experimental.pallas import tpu_sc as plsc

N = 50000
E = 800000
G = 1024
H = 256
OUT = 9

NC = 2
NS = 16
TILE_NODES = 3136
NPAD = NS * TILE_NODES
SUB = 784
NSUB = TILE_NODES // SUB
GROUPS = SUB // 16
EPT = E // NS
ECH = 2000
NCHUNK = EPT // ECH
GP = G + 64
WP = 16


def _prep_body(a_ref, b_ref, c_ref, d_ref, bt_ref,
               sa_ref, da_ref, sb_ref, db_ref,
               sc_ref, dc_ref, sd_ref, dd_ref, bto_ref):
    sa_ref[...] = a_ref[0, :]
    da_ref[...] = a_ref[1, :]
    sb_ref[...] = b_ref[0, :]
    db_ref[...] = b_ref[1, :]
    sc_ref[...] = c_ref[0, :]
    dc_ref[...] = c_ref[1, :]
    sd_ref[...] = d_ref[0, :]
    dd_ref[...] = d_ref[1, :]
    nb = bt_ref.shape[0]
    i = pl.program_id(0)
    gidx = lax.broadcasted_iota(jnp.int32, (nb,), 0) + i * nb
    bto_ref[...] = jnp.where(gidx < N, bt_ref[...], G)


def _sc_body(xe8, xb8, xd8, xw8, xh, batchp,
             sA0, dA0, sB0, dB0, sA1, dA1, sB1, dB1, z8, zp, out,
             accA, accB, pooled, idx0, idx1, dstb, dstb2, rows0, rows1,
             locA, locB, locH, bidx, rowbuf, gsem0, gsem1, ssem0, ssem1):
    c = lax.axis_index("c")
    s = lax.axis_index("s")

    r0 = s * TILE_NODES
    pltpu.sync_copy(z8.at[pl.ds(r0, TILE_NODES)], accA.at[pl.ds(r0, TILE_NODES)])
    pltpu.sync_copy(z8.at[pl.ds(r0, TILE_NODES)], accB.at[pl.ds(r0, TILE_NODES)])

    @pl.when(s == 0)
    def _():
        pltpu.sync_copy(zp, pooled)

    plsc.subcore_barrier()

    ebase = s * EPT

    def _edges(src, dst, table, acc):
        idxs = (idx0, idx1)
        rows = (rows0, rows1)
        dsts = (dstb, dstb2)
        gsems = (gsem0, gsem1)
        ssems = (ssem0, ssem1)
        gd = [None, None]
        sd = [None, None]
        pltpu.sync_copy(src.at[pl.ds(ebase, ECH)], idx0)
        gd[0] = pltpu.async_copy(table.at[idx0], rows0, gsem0)
        for i in range(NCHUNK):
            p = i % 2
            q = 1 - p
            if i + 1 < NCHUNK:
                b = ebase + (i + 1) * ECH
                if sd[q] is not None:
                    sd[q].wait()
                    sd[q] = None
                pltpu.sync_copy(src.at[pl.ds(b, ECH)], idxs[q])
                gd[q] = pltpu.async_copy(table.at[idxs[q]], rows[q], gsems[q])
            gd[p].wait()
            pltpu.sync_copy(dst.at[pl.ds(ebase + i * ECH, ECH)], dsts[p])
            sd[p] = pltpu.async_copy(rows[p], acc.at[dsts[p]], ssems[p], add=True)
        for j in range(2):
            if sd[j] is not None:
                sd[j].wait()

    @pl.when(c == 0)
    def _():
        _edges(sA0, dA0, xe8, accA)
        _edges(sB0, dB0, xb8, accB)

    @pl.when(c == 1)
    def _():
        _edges(sA1, dA1, xd8, accA)
        _edges(sB1, dB1, xw8, accB)

    plsc.subcore_barrier()

    iota16 = lax.iota(jnp.int32, 16)
    colBcnt = jnp.where(c == 0, 3, 4)
    hero_f = jnp.where(c == 0, 0.0, 1.0)
    cnt_f = jnp.where(c == 0, 1.0, 0.0)

    for sub in range(NSUB):
        nb = s * TILE_NODES + sub * SUB
        pltpu.sync_copy(accA.at[pl.ds(nb, SUB)], locA)
        pltpu.sync_copy(accB.at[pl.ds(nb, SUB)], locB)
        pltpu.sync_copy(xh.at[pl.ds(nb, SUB)], locH)
        pltpu.sync_copy(batchp.at[pl.ds(nb, SUB)], bidx)

        def group(j, carry):
            r = iota16 + j * 16
            cA = plsc.load_gather(locA, [r, jnp.full((16,), 4, jnp.int32)])
            cB = plsc.load_gather(locB, [r, jnp.full((16,), colBcnt, jnp.int32)])
            sA = 1.0 / jnp.maximum(cA, 1.0)
            sB = 1.0 / jnp.maximum(cB, 1.0)
            for k in range(4):
                col = jnp.full((16,), k, jnp.int32)
                vA = plsc.load_gather(locA, [r, col])
                plsc.store_scatter(rowbuf, [r, col], vA * sA)
            for k in range(4):
                vB = plsc.load_gather(locB, [r, jnp.full((16,), k, jnp.int32)])
                plsc.store_scatter(rowbuf, [r, jnp.full((16,), 4 + k, jnp.int32)], vB * sB)
            hf = jnp.full((16,), hero_f, jnp.float32)
            for k in range(6):
                vH = plsc.load_gather(locH, [r, jnp.full((16,), k, jnp.int32)])
                plsc.store_scatter(rowbuf, [r, jnp.full((16,), 8 + k, jnp.int32)], vH * hf)
            plsc.store_scatter(rowbuf, [r, jnp.full((16,), 14, jnp.int32)],
                               jnp.full((16,), cnt_f, jnp.float32))
            plsc.store_scatter(rowbuf, [r, jnp.full((16,), 15, jnp.int32)],
                               jnp.zeros((16,), jnp.float32))
            return carry

        lax.fori_loop(0, GROUPS, group, 0)
        pltpu.sync_copy(rowbuf, pooled.at[bidx], add=True)

    plsc.subcore_barrier()

    @pl.when(s == 0)
    def _():
        pltpu.sync_copy(pooled.at[pl.ds(0, G)], out.at[c])


def _tc_body(p0_ref, p1_ref, w0_ref, w1_ref, bt_ref, wfc_ref, bfc_ref, o_ref):
    p0 = p0_ref[...]
    p1 = p1_ref[...]
    lane = lax.broadcasted_iota(jnp.int32, (G, WP), 1)
    cnt = jnp.sum(jnp.where(lane == 14, p0, 0.0), axis=1, keepdims=True)
    scale = 1.0 / jnp.maximum(cnt, 1.0)
    ind = jnp.where(cnt > 0, 1.0, 0.0)
    h = jnp.dot(p0 * scale, w0_ref[...], preferred_element_type=jnp.float32)
    h = h + jnp.dot(p1 * scale, w1_ref[...], preferred_element_type=jnp.float32)
    h = h + ind * bt_ref[...]
    o_ref[...] = jnp.dot(h, wfc_ref[...], preferred_element_type=jnp.float32) + bfc_ref[...]


def kernel(x_hero, x_enemy, x_bullet, x_door, x_wall,
           ei_def, ei_dod, ei_togo, ei_sees, ei_rdef, ei_rdod, ei_rtogo, ei_rsees,
           batch,
           Wl_def, Wr_def, b_def, Wl_dod, Wr_dod, b_dod,
           Wl_togo, Wr_togo, b_togo, Wl_sees, Wr_sees, b_sees,
           Wl_rdef, Wr_rdef, b_rdef, Wl_rdod, Wr_rdod, b_rdod,
           Wl_rtogo, Wr_rtogo, b_rtogo, Wl_rsees, Wr_rsees, b_rsees,
           Wfc, bfc):
    f32 = jnp.float32

    def pad8(x):
        d = x.shape[1]
        return jnp.concatenate(
            [x, jnp.ones((N, 1), f32), jnp.zeros((N, 7 - d), f32)], axis=1)

    EBLK = 65536
    NB = 4096
    prep = pl.pallas_call(
        _prep_body,
        grid=(13,),
        in_specs=[pl.BlockSpec((2, EBLK), lambda i: (0, i))] * 4 + [
            pl.BlockSpec((NB,), lambda i: (i,)),
        ],
        out_specs=[pl.BlockSpec((EBLK,), lambda i: (i,))] * 8 + [
            pl.BlockSpec((NB,), lambda i: (i,)),
        ],
        out_shape=[jax.ShapeDtypeStruct((E,), jnp.int32)] * 8 + [
            jax.ShapeDtypeStruct((NPAD,), jnp.int32),
        ],
    )(ei_rdef, ei_rtogo, ei_rdod, ei_rsees, batch)
    (sA0, dA0, sA1, dA1, sB0, dB0, sB1, dB1, batchp) = prep

    xh = jnp.concatenate([x_hero, jnp.zeros((NPAD - N, 6), f32)], axis=0)
    z8 = jnp.zeros((NPAD, 8), f32)
    zp = jnp.zeros((GP, WP), f32)

    sc_fn = pl.kernel(
        _sc_body,
        out_type=jax.ShapeDtypeStruct((NC, G, WP), f32),
        mesh=plsc.VectorSubcoreMesh(core_axis_name="c", subcore_axis_name="s"),
        compiler_params=pltpu.CompilerParams(
            needs_layout_passes=False, use_tc_tiling_on_sc=False),
        scratch_types=[
            pltpu.VMEM_SHARED((NPAD, 8), f32),
            pltpu.VMEM_SHARED((NPAD, 8), f32),
            pltpu.VMEM_SHARED((GP, WP), f32),
            pltpu.VMEM((ECH,), jnp.int32),
            pltpu.VMEM((ECH,), jnp.int32),
            pltpu.VMEM((ECH,), jnp.int32),
            pltpu.VMEM((ECH,), jnp.int32),
            pltpu.VMEM((ECH, 8), f32),
            pltpu.VMEM((ECH, 8), f32),
            pltpu.VMEM((SUB, 8), f32),
            pltpu.VMEM((SUB, 8), f32),
            pltpu.VMEM((SUB, 6), f32),
            pltpu.VMEM((SUB,), jnp.int32),
            pltpu.VMEM((SUB, WP), f32),
            pltpu.SemaphoreType.DMA,
            pltpu.SemaphoreType.DMA,
            pltpu.SemaphoreType.DMA,
            pltpu.SemaphoreType.DMA,
        ],
    )
    parts = sc_fn(pad8(x_enemy), pad8(x_bullet), pad8(x_door), pad8(x_wall),
                  xh, batchp, sA0, dA0, sB0, dB0, sA1, dA1, sB1, dB1,
                  z8, zp)

    W0 = jnp.concatenate([Wl_rdef, Wl_rdod, jnp.zeros((9, H), f32)], axis=0)
    W1 = jnp.concatenate(
        [Wl_rtogo, Wl_rsees, Wr_rdef + Wr_rdod + Wr_rtogo + Wr_rsees,
         jnp.zeros((2, H), f32)], axis=0)
    btot = (b_rdef + b_rdod + b_rtogo + b_rsees).reshape(1, H)

    q = pl.pallas_call(
        _tc_body,
        out_shape=jax.ShapeDtypeStruct((G, OUT), f32),
    )(parts[0], parts[1], W0, W1, btot, Wfc, bfc.reshape(1, OUT))
    return q


# speedup vs baseline: 1.2396x; 1.0003x over previous
import jax
import jax.numpy as jnp
from jax import lax
from jax.experimental import pallas as pl
from jax.experimental.pallas import tpu as pltpu
from jax.experimental.pallas import tpu_sc as plsc

N = 50000
E = 800000
G = 1024
H = 256
OUT = 9

NC = 2
NS = 16
TILE_NODES = 3136
NPAD = NS * TILE_NODES
SUB = 784
NSUB = TILE_NODES // SUB
GROUPS = SUB // 16
EPT = E // NS
ECH = 2000
NCHUNK = EPT // ECH
GP = G + 64
WP = 16


def _prep_body(a_ref, b_ref, c_ref, d_ref, bt_ref,
               sa_ref, da_ref, sb_ref, db_ref,
               sc_ref, dc_ref, sd_ref, dd_ref, bto_ref):
    sa_ref[...] = a_ref[0, :]
    da_ref[...] = a_ref[1, :]
    sb_ref[...] = b_ref[0, :]
    db_ref[...] = b_ref[1, :]
    sc_ref[...] = c_ref[0, :]
    dc_ref[...] = c_ref[1, :]
    sd_ref[...] = d_ref[0, :]
    dd_ref[...] = d_ref[1, :]
    nb = bt_ref.shape[0]
    i = pl.program_id(0)
    gidx = lax.broadcasted_iota(jnp.int32, (nb,), 0) + i * nb
    bto_ref[...] = jnp.where(gidx < N, bt_ref[...], G)


def _sc_body(xe8, xb8, xd8, xw8, xh, batchp,
             sA0, dA0, sB0, dB0, sA1, dA1, sB1, dB1, z8, zp, out,
             accA, accB, pooled, idx0, idx1, dstb, rows0, rows1,
             locA, locB, locH, bidx, rowbuf, gsem0, gsem1):
    c = lax.axis_index("c")
    s = lax.axis_index("s")

    r0 = s * TILE_NODES
    pltpu.sync_copy(z8.at[pl.ds(r0, TILE_NODES)], accA.at[pl.ds(r0, TILE_NODES)])
    pltpu.sync_copy(z8.at[pl.ds(r0, TILE_NODES)], accB.at[pl.ds(r0, TILE_NODES)])

    @pl.when(s == 0)
    def _():
        pltpu.sync_copy(zp, pooled)

    plsc.subcore_barrier()

    ebase = s * EPT

    def _edges(src, dst, table, acc):
        idxs = (idx0, idx1)
        rows = (rows0, rows1)
        gsems = (gsem0, gsem1)
        desc = [None, None]
        pltpu.sync_copy(src.at[pl.ds(ebase, ECH)], idx0)
        desc[0] = pltpu.async_copy(table.at[idx0], rows0, gsem0)
        for i in range(NCHUNK):
            p = i % 2
            q = 1 - p
            if i + 1 < NCHUNK:
                b = ebase + (i + 1) * ECH
                pltpu.sync_copy(src.at[pl.ds(b, ECH)], idxs[q])
                desc[q] = pltpu.async_copy(table.at[idxs[q]], rows[q], gsems[q])
            desc[p].wait()
            pltpu.sync_copy(dst.at[pl.ds(ebase + i * ECH, ECH)], dstb)
            pltpu.sync_copy(rows[p], acc.at[dstb], add=True)

    @pl.when(c == 0)
    def _():
        _edges(sA0, dA0, xe8, accA)
        _edges(sB0, dB0, xb8, accB)

    @pl.when(c == 1)
    def _():
        _edges(sA1, dA1, xd8, accA)
        _edges(sB1, dB1, xw8, accB)

    plsc.subcore_barrier()

    iota16 = lax.iota(jnp.int32, 16)
    colBcnt = jnp.where(c == 0, 3, 4)
    hero_f = jnp.where(c == 0, 0.0, 1.0)
    cnt_f = jnp.where(c == 0, 1.0, 0.0)

    for sub in range(NSUB):
        nb = s * TILE_NODES + sub * SUB
        pltpu.sync_copy(accA.at[pl.ds(nb, SUB)], locA)
        pltpu.sync_copy(accB.at[pl.ds(nb, SUB)], locB)
        pltpu.sync_copy(xh.at[pl.ds(nb, SUB)], locH)
        pltpu.sync_copy(batchp.at[pl.ds(nb, SUB)], bidx)

        def group(j, carry):
            r = iota16 + j * 16
            cA = plsc.load_gather(locA, [r, jnp.full((16,), 4, jnp.int32)])
            cB = plsc.load_gather(locB, [r, jnp.full((16,), colBcnt, jnp.int32)])
            sA = 1.0 / jnp.maximum(cA, 1.0)
            sB = 1.0 / jnp.maximum(cB, 1.0)
            for k in range(4):
                col = jnp.full((16,), k, jnp.int32)
                vA = plsc.load_gather(locA, [r, col])
                plsc.store_scatter(rowbuf, [r, col], vA * sA)
            for k in range(4):
                vB = plsc.load_gather(locB, [r, jnp.full((16,), k, jnp.int32)])
                plsc.store_scatter(rowbuf, [r, jnp.full((16,), 4 + k, jnp.int32)], vB * sB)
            hf = jnp.full((16,), hero_f, jnp.float32)
            for k in range(6):
                vH = plsc.load_gather(locH, [r, jnp.full((16,), k, jnp.int32)])
                plsc.store_scatter(rowbuf, [r, jnp.full((16,), 8 + k, jnp.int32)], vH * hf)
            plsc.store_scatter(rowbuf, [r, jnp.full((16,), 14, jnp.int32)],
                               jnp.full((16,), cnt_f, jnp.float32))
            plsc.store_scatter(rowbuf, [r, jnp.full((16,), 15, jnp.int32)],
                               jnp.zeros((16,), jnp.float32))
            return carry

        lax.fori_loop(0, GROUPS, group, 0)
        pltpu.sync_copy(rowbuf, pooled.at[bidx], add=True)

    plsc.subcore_barrier()

    @pl.when(s == 0)
    def _():
        pltpu.sync_copy(pooled.at[pl.ds(0, G)], out.at[c])


def _tc_body(p0_ref, p1_ref, w0_ref, w1_ref, bt_ref, wfc_ref, bfc_ref, o_ref):
    p0 = p0_ref[...]
    p1 = p1_ref[...]
    lane = lax.broadcasted_iota(jnp.int32, (G, WP), 1)
    cnt = jnp.sum(jnp.where(lane == 14, p0, 0.0), axis=1, keepdims=True)
    scale = 1.0 / jnp.maximum(cnt, 1.0)
    ind = jnp.where(cnt > 0, 1.0, 0.0)
    h = jnp.dot(p0 * scale, w0_ref[...], preferred_element_type=jnp.float32)
    h = h + jnp.dot(p1 * scale, w1_ref[...], preferred_element_type=jnp.float32)
    h = h + ind * bt_ref[...]
    o_ref[...] = jnp.dot(h, wfc_ref[...], preferred_element_type=jnp.float32) + bfc_ref[...]


def kernel(x_hero, x_enemy, x_bullet, x_door, x_wall,
           ei_def, ei_dod, ei_togo, ei_sees, ei_rdef, ei_rdod, ei_rtogo, ei_rsees,
           batch,
           Wl_def, Wr_def, b_def, Wl_dod, Wr_dod, b_dod,
           Wl_togo, Wr_togo, b_togo, Wl_sees, Wr_sees, b_sees,
           Wl_rdef, Wr_rdef, b_rdef, Wl_rdod, Wr_rdod, b_rdod,
           Wl_rtogo, Wr_rtogo, b_rtogo, Wl_rsees, Wr_rsees, b_rsees,
           Wfc, bfc):
    f32 = jnp.float32

    def pad8(x):
        d = x.shape[1]
        return jnp.concatenate(
            [x, jnp.ones((N, 1), f32), jnp.zeros((N, 7 - d), f32)], axis=1)

    EBLK = 65536
    NB = 4096
    prep = pl.pallas_call(
        _prep_body,
        grid=(13,),
        in_specs=[pl.BlockSpec((2, EBLK), lambda i: (0, i))] * 4 + [
            pl.BlockSpec((NB,), lambda i: (i,)),
        ],
        out_specs=[pl.BlockSpec((EBLK,), lambda i: (i,))] * 8 + [
            pl.BlockSpec((NB,), lambda i: (i,)),
        ],
        out_shape=[jax.ShapeDtypeStruct((E,), jnp.int32)] * 8 + [
            jax.ShapeDtypeStruct((NPAD,), jnp.int32),
        ],
    )(ei_rdef, ei_rtogo, ei_rdod, ei_rsees, batch)
    (sA0, dA0, sA1, dA1, sB0, dB0, sB1, dB1, batchp) = prep

    xh = jnp.concatenate([x_hero, jnp.zeros((NPAD - N, 6), f32)], axis=0)
    z8 = jnp.zeros((NPAD, 8), f32)
    zp = jnp.zeros((GP, WP), f32)

    sc_fn = pl.kernel(
        _sc_body,
        out_type=jax.ShapeDtypeStruct((NC, G, WP), f32),
        mesh=plsc.VectorSubcoreMesh(core_axis_name="c", subcore_axis_name="s"),
        compiler_params=pltpu.CompilerParams(
            needs_layout_passes=False, use_tc_tiling_on_sc=False),
        scratch_types=[
            pltpu.VMEM_SHARED((NPAD, 8), f32),
            pltpu.VMEM_SHARED((NPAD, 8), f32),
            pltpu.VMEM_SHARED((GP, WP), f32),
            pltpu.VMEM((ECH,), jnp.int32),
            pltpu.VMEM((ECH,), jnp.int32),
            pltpu.VMEM((ECH,), jnp.int32),
            pltpu.VMEM((ECH, 8), f32),
            pltpu.VMEM((ECH, 8), f32),
            pltpu.VMEM((SUB, 8), f32),
            pltpu.VMEM((SUB, 8), f32),
            pltpu.VMEM((SUB, 6), f32),
            pltpu.VMEM((SUB,), jnp.int32),
            pltpu.VMEM((SUB, WP), f32),
            pltpu.SemaphoreType.DMA,
            pltpu.SemaphoreType.DMA,
        ],
    )
    parts = sc_fn(pad8(x_enemy), pad8(x_bullet), pad8(x_door), pad8(x_wall),
                  xh, batchp, sA0, dA0, sB0, dB0, sA1, dA1, sB1, dB1,
                  z8, zp)

    W0 = jnp.concatenate([Wl_rdef, Wl_rdod, jnp.zeros((9, H), f32)], axis=0)
    W1 = jnp.concatenate(
        [Wl_rtogo, Wl_rsees, Wr_rdef + Wr_rdod + Wr_rtogo + Wr_rsees,
         jnp.zeros((2, H), f32)], axis=0)
    btot = (b_rdef + b_rdod + b_rtogo + b_rsees).reshape(1, H)

    q = pl.pallas_call(
        _tc_body,
        out_shape=jax.ShapeDtypeStruct((G, OUT), f32),
    )(parts[0], parts[1], W0, W1, btot, Wfc, bfc.reshape(1, OUT))
    return q
